# baseline (speedup 1.0000x reference)
import functools

import jax
import jax.numpy as jnp
from jax import lax
from jax.experimental import pallas as pl
from jax.experimental.pallas import tpu as pltpu
from jax.experimental.pallas import tpu_sc as plsc

N_NODES = 10000
N_EDGES = 320000
D = 128
D_EDGE = 16

NC = 2
NS = 16
NW = NC * NS
CH = 40
EDGES_PER_W = N_EDGES // NW
CHUNKS = EDGES_PER_W // CH
N_PAD = 10240
ROWS_PER_TEC = N_PAD // NS
FULL_FLUSHES = ROWS_PER_TEC // CH
IB = 25
NB = CHUNKS // IB

_GB = 2000
_EB = 6400
_FB = 2000


def _g_body(x_ref, wx_ref, bx_ref, wm1_ref, g_ref):
    h = jnp.dot(x_ref[...], wx_ref[...], preferred_element_type=jnp.float32)
    h = h + bx_ref[...]
    g_ref[...] = jnp.dot(h, wm1_ref[...], preferred_element_type=jnp.float32)


def _compute_g(x, Wx, bx2, Wm1):
    return pl.pallas_call(
        _g_body,
        grid=(N_NODES // _GB,),
        in_specs=[
            pl.BlockSpec((_GB, D), lambda i: (i, 0)),
            pl.BlockSpec((D, D), lambda i: (0, 0)),
            pl.BlockSpec((1, D), lambda i: (0, 0)),
            pl.BlockSpec((D, D), lambda i: (0, 0)),
        ],
        out_specs=pl.BlockSpec((_GB, D), lambda i: (i, 0)),
        out_shape=jax.ShapeDtypeStruct((N_NODES, D), jnp.float32),
    )(x, Wx, bx2, Wm1)


def _p_body(et_ref, we_ref, be_ref, wm2_ref, bm_ref, p_ref):
    w2 = jnp.dot(we_ref[...], wm2_ref[...], preferred_element_type=jnp.float32)
    c2 = jnp.dot(be_ref[...], wm2_ref[...], preferred_element_type=jnp.float32)
    c2 = c2 + bm_ref[...]
    p = lax.dot_general(et_ref[...], w2,
                        dimension_numbers=(((0,), (0,)), ((), ())),
                        preferred_element_type=jnp.float32)
    p_ref[...] = p + c2


def _compute_p(edge_attr_t, We, be2, Wm2, bm2):
    return pl.pallas_call(
        _p_body,
        grid=(N_EDGES // _EB,),
        in_specs=[
            pl.BlockSpec((D_EDGE, _EB), lambda i: (0, i)),
            pl.BlockSpec((D_EDGE, D), lambda i: (0, 0)),
            pl.BlockSpec((1, D), lambda i: (0, 0)),
            pl.BlockSpec((D, D), lambda i: (0, 0)),
            pl.BlockSpec((1, D), lambda i: (0, 0)),
        ],
        out_specs=pl.BlockSpec((_EB, D), lambda i: (i, 0)),
        out_shape=jax.ShapeDtypeStruct((N_EDGES, D), jnp.float32),
    )(edge_attr_t, We, be2, Wm2, bm2)


_MESH = plsc.VectorSubcoreMesh(
    core_axis_name="c", subcore_axis_name="s", num_cores=NC, num_subcores=NS
)


@functools.partial(
    pl.kernel,
    out_type=jax.ShapeDtypeStruct((NC * N_PAD, D), jnp.float32),
    mesh=_MESH,
    scratch_types=[
        pltpu.VMEM((IB, CH), jnp.int32),
        pltpu.VMEM((IB, CH), jnp.int32),
        pltpu.VMEM((CH, D), jnp.float32),
        pltpu.VMEM((CH, D), jnp.float32),
        pltpu.VMEM((CH, D), jnp.float32),
        pltpu.VMEM((CH, D), jnp.float32),
        pltpu.VMEM_SHARED((N_PAD, D), jnp.float32),
        pltpu.SemaphoreType.DMA,
        pltpu.SemaphoreType.DMA,
    ],
)
def _sc_aggregate(g_hbm, p_hbm, eidx_hbm, out_hbm,
                  sidx, didx, gbuf0, gbuf1, pbuf0, pbuf1, acc, sem0, sem1):
    cid = lax.axis_index("c")
    sid = lax.axis_index("s")
    wid = sid * NC + cid
    base = sid * ROWS_PER_TEC

    zero = jnp.zeros((16,), jnp.float32)

    def _zrow(r, carry):
        for c in range(D // 16):
            gbuf0[r, pl.ds(c * 16, 16)] = zero
        return carry

    lax.fori_loop(0, CH, _zrow, 0)

    def _zcopy(i, carry):
        pltpu.sync_copy(gbuf0, acc.at[pl.ds(base + i * CH, CH)])
        return carry

    lax.fori_loop(0, FULL_FLUSHES, _zcopy, 0)
    plsc.subcore_barrier()

    def _issue(ebase_blk, j, gbuf, pbuf, sem):
        pltpu.async_copy(g_hbm.at[sidx.at[j]], gbuf, sem)
        pltpu.async_copy(p_hbm.at[pl.ds(ebase_blk + j * CH, CH)], pbuf, sem)

    def _drain(gbuf, pbuf, sem):
        pltpu.make_async_copy(p_hbm.at[pl.ds(0, CH)], gbuf, sem).wait()
        pltpu.make_async_copy(p_hbm.at[pl.ds(0, CH)], pbuf, sem).wait()

    def _compute_scatter(j, gbuf, pbuf):
        @plsc.parallel_loop(0, CH, step=1, unroll=1)
        def _row(r):
            for c in range(D // 16):
                sl = pl.ds(c * 16, 16)
                m = gbuf[r, sl] + pbuf[r, sl]
                pbuf[r, sl] = jnp.maximum(m, m * jnp.float32(0.01))

        pltpu.sync_copy(pbuf, acc.at[didx.at[j]], add=True)

    def _block(b, carry):
        pltpu.sync_copy(eidx_hbm.at[wid * NB + b], sidx)
        pltpu.sync_copy(eidx_hbm.at[NW * NB + wid * NB + b], didx)
        ebase_blk = wid * EDGES_PER_W + b * IB * CH
        _issue(ebase_blk, 0, gbuf0, pbuf0, sem0)

        def _pair(i, c2):
            j0 = 2 * i
            _issue(ebase_blk, j0 + 1, gbuf1, pbuf1, sem1)
            _drain(gbuf0, pbuf0, sem0)
            _compute_scatter(j0, gbuf0, pbuf0)
            _issue(ebase_blk, j0 + 2, gbuf0, pbuf0, sem0)
            _drain(gbuf1, pbuf1, sem1)
            _compute_scatter(j0 + 1, gbuf1, pbuf1)
            return c2

        lax.fori_loop(0, (IB - 1) // 2, _pair, 0)
        _drain(gbuf0, pbuf0, sem0)
        _compute_scatter(IB - 1, gbuf0, pbuf0)
        return carry

    lax.fori_loop(0, NB, _block, 0)
    plsc.subcore_barrier()

    obase = cid * N_PAD + base

    def _flush(i, carry):
        pltpu.sync_copy(acc.at[pl.ds(base + i * CH, CH)], gbuf0)
        pltpu.sync_copy(gbuf0, out_hbm.at[pl.ds(obase + i * CH, CH)])
        return carry

    lax.fori_loop(0, FULL_FLUSHES, _flush, 0)


def _f_body(p_ref, beta_ref, o_ref):
    s = p_ref[0] + p_ref[1]
    b = jnp.maximum(beta_ref[0, 0], jnp.float32(0.0))
    o_ref[...] = jax.nn.sigmoid(s) * b


def _finalize(parts, beta2):
    return pl.pallas_call(
        _f_body,
        grid=(N_NODES // _FB,),
        in_specs=[
            pl.BlockSpec((NC, _FB, D), lambda i: (0, i, 0)),
            pl.BlockSpec(memory_space=pltpu.SMEM),
        ],
        out_specs=pl.BlockSpec((_FB, D), lambda i: (i, 0)),
        out_shape=jax.ShapeDtypeStruct((N_NODES, D), jnp.float32),
    )(parts, beta2)


def kernel(x, edge_index, edge_attr, Wx, bx, We, be, Wm, bm, beta):
    Wm1 = Wm[:D]
    Wm2 = Wm[D:]
    eidx = edge_index.reshape(2 * NW * NB, IB, CH)
    g = _compute_g(x, Wx, bx.reshape(1, D), Wm1)
    p = _compute_p(edge_attr.T, We, be.reshape(1, D), Wm2, bm.reshape(1, D))
    parts = _sc_aggregate(g, p, eidx)
    parts = parts.reshape(NC, N_PAD, D)
    return _finalize(parts, beta.reshape(1, 1))


# speedup vs baseline: 1.2348x; 1.2348x over previous
import functools

import jax
import jax.numpy as jnp
from jax import lax
from jax.experimental import pallas as pl
from jax.experimental.pallas import tpu as pltpu
from jax.experimental.pallas import tpu_sc as plsc

N_NODES = 10000
N_EDGES = 320000
D = 128
D_EDGE = 16

NC = 2
NS = 16
NW = NC * NS
CH = 80
EDGES_PER_W = N_EDGES // NW
CHUNKS = EDGES_PER_W // CH
N_PAD = 10240
ROWS_PER_TEC = N_PAD // NS
FULL_FLUSHES = ROWS_PER_TEC // CH
IB = 25
NB = CHUNKS // IB

_GB = 2000
_EB = 12800
_FB = 2000


def _g_body(x_ref, wx_ref, bx_ref, wm1_ref, g_ref):
    h = jnp.dot(x_ref[...], wx_ref[...], preferred_element_type=jnp.float32)
    h = h + bx_ref[...]
    g_ref[...] = jnp.dot(h, wm1_ref[...], preferred_element_type=jnp.float32)


def _compute_g(x, Wx, bx2, Wm1):
    return pl.pallas_call(
        _g_body,
        grid=(N_NODES // _GB,),
        in_specs=[
            pl.BlockSpec((_GB, D), lambda i: (i, 0)),
            pl.BlockSpec((D, D), lambda i: (0, 0)),
            pl.BlockSpec((1, D), lambda i: (0, 0)),
            pl.BlockSpec((D, D), lambda i: (0, 0)),
        ],
        out_specs=pl.BlockSpec((_GB, D), lambda i: (i, 0)),
        out_shape=jax.ShapeDtypeStruct((N_NODES, D), jnp.float32),
    )(x, Wx, bx2, Wm1)


def _p_body(et_ref, we_ref, be_ref, wm2_ref, bm_ref, p_ref):
    w2 = jnp.dot(we_ref[...], wm2_ref[...], preferred_element_type=jnp.float32)
    c2 = jnp.dot(be_ref[...], wm2_ref[...], preferred_element_type=jnp.float32)
    c2 = c2 + bm_ref[...]
    p = lax.dot_general(et_ref[...], w2,
                        dimension_numbers=(((0,), (0,)), ((), ())),
                        preferred_element_type=jnp.float32)
    p_ref[...] = p + c2


def _compute_p(edge_attr_t, We, be2, Wm2, bm2):
    return pl.pallas_call(
        _p_body,
        grid=(N_EDGES // _EB,),
        in_specs=[
            pl.BlockSpec((D_EDGE, _EB), lambda i: (0, i)),
            pl.BlockSpec((D_EDGE, D), lambda i: (0, 0)),
            pl.BlockSpec((1, D), lambda i: (0, 0)),
            pl.BlockSpec((D, D), lambda i: (0, 0)),
            pl.BlockSpec((1, D), lambda i: (0, 0)),
        ],
        out_specs=pl.BlockSpec((_EB, D), lambda i: (i, 0)),
        out_shape=jax.ShapeDtypeStruct((N_EDGES, D), jnp.float32),
    )(edge_attr_t, We, be2, Wm2, bm2)


_MESH = plsc.VectorSubcoreMesh(
    core_axis_name="c", subcore_axis_name="s", num_cores=NC, num_subcores=NS
)


@functools.partial(
    pl.kernel,
    out_type=jax.ShapeDtypeStruct((NC * N_PAD, D), jnp.float32),
    mesh=_MESH,
    scratch_types=[
        pltpu.VMEM((IB, CH), jnp.int32),
        pltpu.VMEM((IB, CH), jnp.int32),
        pltpu.VMEM((CH, D), jnp.float32),
        pltpu.VMEM((CH, D), jnp.float32),
        pltpu.VMEM((CH, D), jnp.float32),
        pltpu.VMEM((CH, D), jnp.float32),
        pltpu.VMEM_SHARED((N_PAD, D), jnp.float32),
        pltpu.SemaphoreType.DMA,
        pltpu.SemaphoreType.DMA,
    ],
)
def _sc_aggregate(g_hbm, p_hbm, eidx_hbm, out_hbm,
                  sidx, didx, gbuf0, gbuf1, pbuf0, pbuf1, acc, sem0, sem1):
    cid = lax.axis_index("c")
    sid = lax.axis_index("s")
    wid = sid * NC + cid
    base = sid * ROWS_PER_TEC

    zero = jnp.zeros((16,), jnp.float32)

    def _zrow(r, carry):
        for c in range(D // 16):
            gbuf0[r, pl.ds(c * 16, 16)] = zero
        return carry

    lax.fori_loop(0, CH, _zrow, 0)

    def _zcopy(i, carry):
        pltpu.sync_copy(gbuf0, acc.at[pl.ds(base + i * CH, CH)])
        return carry

    lax.fori_loop(0, FULL_FLUSHES, _zcopy, 0)
    plsc.subcore_barrier()

    def _issue(ebase_blk, j, gbuf, pbuf, sem):
        pltpu.async_copy(g_hbm.at[sidx.at[j]], gbuf, sem)
        pltpu.async_copy(p_hbm.at[pl.ds(ebase_blk + j * CH, CH)], pbuf, sem)

    def _drain(gbuf, pbuf, sem):
        pltpu.make_async_copy(p_hbm.at[pl.ds(0, CH)], gbuf, sem).wait()
        pltpu.make_async_copy(p_hbm.at[pl.ds(0, CH)], pbuf, sem).wait()

    def _compute_scatter(j, gbuf, pbuf):
        @plsc.parallel_loop(0, CH, step=1, unroll=1)
        def _row(r):
            for c in range(D // 16):
                sl = pl.ds(c * 16, 16)
                m = gbuf[r, sl] + pbuf[r, sl]
                pbuf[r, sl] = jnp.maximum(m, m * jnp.float32(0.01))

        pltpu.sync_copy(pbuf, acc.at[didx.at[j]], add=True)

    def _block(b, carry):
        pltpu.sync_copy(eidx_hbm.at[wid * NB + b], sidx)
        pltpu.sync_copy(eidx_hbm.at[NW * NB + wid * NB + b], didx)
        ebase_blk = wid * EDGES_PER_W + b * IB * CH
        _issue(ebase_blk, 0, gbuf0, pbuf0, sem0)

        def _pair(i, c2):
            j0 = 2 * i
            _issue(ebase_blk, j0 + 1, gbuf1, pbuf1, sem1)
            _drain(gbuf0, pbuf0, sem0)
            _compute_scatter(j0, gbuf0, pbuf0)
            _issue(ebase_blk, j0 + 2, gbuf0, pbuf0, sem0)
            _drain(gbuf1, pbuf1, sem1)
            _compute_scatter(j0 + 1, gbuf1, pbuf1)
            return c2

        lax.fori_loop(0, (IB - 1) // 2, _pair, 0)
        _drain(gbuf0, pbuf0, sem0)
        _compute_scatter(IB - 1, gbuf0, pbuf0)
        return carry

    lax.fori_loop(0, NB, _block, 0)
    plsc.subcore_barrier()

    obase = cid * N_PAD + base

    def _flush(i, carry):
        pltpu.sync_copy(acc.at[pl.ds(base + i * CH, CH)], gbuf0)
        pltpu.sync_copy(gbuf0, out_hbm.at[pl.ds(obase + i * CH, CH)])
        return carry

    lax.fori_loop(0, FULL_FLUSHES, _flush, 0)


def _f_body(p_ref, beta_ref, o_ref):
    s = p_ref[0] + p_ref[1]
    b = jnp.maximum(beta_ref[0, 0], jnp.float32(0.0))
    o_ref[...] = jax.nn.sigmoid(s) * b


def _finalize(parts, beta2):
    return pl.pallas_call(
        _f_body,
        grid=(N_NODES // _FB,),
        in_specs=[
            pl.BlockSpec((NC, _FB, D), lambda i: (0, i, 0)),
            pl.BlockSpec(memory_space=pltpu.SMEM),
        ],
        out_specs=pl.BlockSpec((_FB, D), lambda i: (i, 0)),
        out_shape=jax.ShapeDtypeStruct((N_NODES, D), jnp.float32),
    )(parts, beta2)


def kernel(x, edge_index, edge_attr, Wx, bx, We, be, Wm, bm, beta):
    Wm1 = Wm[:D]
    Wm2 = Wm[D:]
    eidx = edge_index.reshape(2 * NW * NB, IB, CH)
    g = _compute_g(x, Wx, bx.reshape(1, D), Wm1)
    p = _compute_p(edge_attr.T, We, be.reshape(1, D), Wm2, bm.reshape(1, D))
    parts = _sc_aggregate(g, p, eidx)
    parts = parts.reshape(NC, N_PAD, D)
    return _finalize(parts, beta.reshape(1, 1))
